# baseline (speedup 1.0000x reference)
import jax
import jax.numpy as jnp
from jax.experimental import pallas as pl

NUM_CLASSES = 1000
BATCH = 16384
BLK = 512


def _onehot_body(x_ref, o_ref):
    xv = x_ref[...]
    iota = jax.lax.broadcasted_iota(jnp.int32, (BLK, NUM_CLASSES), 1)
    o_ref[...] = (xv[:, None] == iota).astype(jnp.float32)


def kernel(x):
    grid = (BATCH // BLK,)
    return pl.pallas_call(
        _onehot_body,
        grid=grid,
        in_specs=[pl.BlockSpec((BLK,), lambda i: (i,))],
        out_specs=pl.BlockSpec((BLK, NUM_CLASSES), lambda i: (i, 0)),
        out_shape=jax.ShapeDtypeStruct((BATCH, NUM_CLASSES), jnp.float32),
    )(x)
